# Initial kernel scaffold; baseline (speedup 1.0000x reference)
import jax
import jax.numpy as jnp
from jax.experimental import pallas as pl


def kernel(x, edge_index, W_in, b_in, W0, asrc0, adst0, b0, W1, asrc1, adst1, b1, W2, asrc2, adst2, b2):
    raise NotImplementedError("write your pallas kernel here")


# speedup vs baseline: 70.5658x; 70.5658x over previous
import jax
import jax.numpy as jnp
from jax import lax
from jax.experimental import pallas as pl
from jax.experimental.pallas import tpu as pltpu
from jax.experimental.pallas import tpu_sc as plsc

N = 100000
E = 1600000
NT = 32
NSUB = 16
NB = 64
BSZ = 1568
EPT = E // NT
DIV_M = 2675
DIV_S = 17
EPAD = E + 1024
SUP = 512
TCR = 1000

_mesh = plsc.VectorSubcoreMesh(core_axis_name="c", subcore_axis_name="s")
_sc_params = pltpu.CompilerParams(needs_layout_passes=False,
                                  use_tc_tiling_on_sc=False)

_i32 = jnp.int32
_f32 = jnp.float32


def _full(v):
    return jnp.full((16,), v, _i32)


def _leaky(t):
    return jnp.where(t > 0, t, 0.2 * t)


def _bucket(d16):
    return ((d16 >> 5) * DIV_M) >> DIV_S


def _p1_body(dst_hbm, cnt_hbm, hist, dbuf):
    c = lax.axis_index("c")
    s = lax.axis_index("s")
    w = c * NSUB + s
    iota = lax.iota(_i32, 16)
    for b in range(NB):
        hist[b, :] = jnp.zeros((16,), _i32)
    ones = jnp.ones((16,), _i32)

    def chunk(k, carry):
        pltpu.sync_copy(dst_hbm.at[pl.ds(w * EPT + k * 2000, 2000)], dbuf)

        def grp(g, cc):
            d16 = plsc.load_gather(dbuf, [g * 16 + iota])
            plsc.addupdate_scatter(hist, [_bucket(d16), iota], ones)
            return cc

        return lax.fori_loop(0, 125, grp, carry)

    lax.fori_loop(0, 25, chunk, 0)
    pltpu.sync_copy(hist, cnt_hbm.at[w])


_p1 = pl.kernel(
    _p1_body,
    out_type=jax.ShapeDtypeStruct((NT, NB, 16), _i32),
    mesh=_mesh,
    compiler_params=_sc_params,
    scratch_types=[
        pltpu.VMEM((NB, 16), _i32),
        pltpu.VMEM((2000,), _i32),
    ],
)


def _p2_body(src_hbm, dst_hbm, cnt_hbm, part_hbm, bases_hbm,
             cntbuf, offtab, basesbuf, sbuf, dbuf, valbuf, posbuf, sem):
    c = lax.axis_index("c")
    s = lax.axis_index("s")
    w = c * NSUB + s
    iota = lax.iota(_i32, 16)
    pltpu.sync_copy(cnt_hbm, cntbuf)
    for b in range(9):
        basesbuf[b * 16:(b + 1) * 16] = jnp.zeros((16,), _i32)

    def off_step(g, carry):
        b = g >> 5
        wp = g & 31
        carry = jnp.where(wp == 0, (carry + 7) & ~7, carry)
        v = plsc.load_gather(cntbuf, [_full(wp), _full(b), iota])
        cs = plsc.cumsum(v)
        tot = cs[15]
        excl = carry + (cs - v)
        plsc.store_scatter(offtab, [_full(b), iota], excl,
                           mask=jnp.broadcast_to(wp == w, (16,)))
        plsc.store_scatter(basesbuf, [_full(b)], _full(carry),
                           mask=(iota == 0) & (wp == 0))
        plsc.store_scatter(basesbuf, [_full(NB + b)], _full(carry + tot),
                           mask=(iota == 0) & (wp == 31))
        return carry + tot

    lax.fori_loop(0, NB * 32, off_step, jnp.int32(0))

    @pl.when((c == 0) & (s == 0))
    def _():
        pltpu.sync_copy(basesbuf, bases_hbm)

    def batch(bi, carry):
        base = w * EPT + bi * 128
        nreal = jnp.where(bi < 390, 128, 80)
        pltpu.sync_copy(src_hbm.at[pl.ds(base, 128)], sbuf)
        pltpu.sync_copy(dst_hbm.at[pl.ds(base, 128)], dbuf)

        def grp(g, cc):
            p16 = g * 16 + iota
            real = p16 < nreal
            s16 = plsc.load_gather(sbuf, [p16])
            d16 = plsc.load_gather(dbuf, [p16])
            d16 = jnp.clip(d16, 0, N - 1)
            b = _bucket(d16)
            dl = d16 - b * BSZ
            val = jnp.where(real, s16 | jnp.left_shift(dl, 17), 0)
            pos = plsc.load_gather(offtab, [b, iota])
            plsc.store_scatter(offtab, [b, iota], pos + 1, mask=real)
            pos = jnp.where(real, pos, EPAD - 128 + p16)
            plsc.store_scatter(valbuf, [p16], val)
            plsc.store_scatter(posbuf, [_full(0), p16], pos)
            return cc

        lax.fori_loop(0, 8, grp, carry)
        pltpu.async_copy(valbuf, part_hbm.at[posbuf.at[0]], sem).wait()
        return carry

    lax.fori_loop(0, 391, batch, 0)


_p2 = pl.kernel(
    _p2_body,
    out_type=(
        jax.ShapeDtypeStruct((EPAD,), _i32),
        jax.ShapeDtypeStruct((144,), _i32),
    ),
    mesh=_mesh,
    compiler_params=_sc_params,
    scratch_types=[
        pltpu.VMEM((NT, NB, 16), _i32),
        pltpu.VMEM((NB, 16), _i32),
        pltpu.VMEM((144,), _i32),
        pltpu.VMEM((128,), _i32),
        pltpu.VMEM((128,), _i32),
        pltpu.VMEM((128,), _i32),
        pltpu.VMEM((1, 128), _i32),
        pltpu.SemaphoreType.DMA,
    ],
)


def _tc_prep_common(h, w_ref, a_ref, d_ref, hprow_ref, ald_ref, gmax_ref, i):
    hp = jnp.dot(h, w_ref[...], preferred_element_type=_f32)
    als = jnp.dot(hp, a_ref[...], preferred_element_type=_f32)
    ald = jnp.dot(hp, d_ref[...], preferred_element_type=_f32)
    fout = hp.shape[1]
    h_ = als.shape[1]
    rw = hprow_ref.shape[1]
    pad = jnp.zeros((hp.shape[0], rw - fout - h_), _f32)
    hprow_ref[...] = jnp.concatenate([hp, als, pad], axis=1)
    ald_ref[...] = ald
    bm = jnp.max(als, axis=0, keepdims=True)

    @pl.when(i == 0)
    def _():
        gmax_ref[...] = bm

    @pl.when(i > 0)
    def _():
        gmax_ref[...] = jnp.maximum(gmax_ref[...], bm)


def _tc0_body(x_ref, win_ref, bin_ref, w_ref, a_ref, d_ref,
              hprow_ref, ald_ref, gmax_ref):
    i = pl.program_id(0)
    t = x_ref[...] * win_ref[...] + bin_ref[...]
    h = jnp.where(t > 0, t, jnp.exp(t) - 1.0)
    _tc_prep_common(h, w_ref, a_ref, d_ref, hprow_ref, ald_ref, gmax_ref, i)


def _tc_body(h_ref, w_ref, a_ref, d_ref, hprow_ref, ald_ref, gmax_ref):
    i = pl.program_id(0)
    _tc_prep_common(h_ref[...], w_ref, a_ref, d_ref,
                    hprow_ref, ald_ref, gmax_ref, i)


def _make_tc(fin, fout, nh, rw, first):
    body = _tc0_body if first else _tc_body
    in_specs = [pl.BlockSpec((TCR, fin), lambda i: (i, 0))]
    kdim = 16 if first else fin
    if first:
        in_specs += [pl.BlockSpec((1, 16), lambda i: (0, 0)),
                     pl.BlockSpec((1, 16), lambda i: (0, 0))]
    in_specs += [
        pl.BlockSpec((kdim, fout), lambda i: (0, 0)),
        pl.BlockSpec((fout, nh), lambda i: (0, 0)),
        pl.BlockSpec((fout, nh), lambda i: (0, 0)),
    ]
    return pl.pallas_call(
        body,
        grid=(N // TCR,),
        in_specs=in_specs,
        out_specs=[
            pl.BlockSpec((TCR, rw), lambda i: (i, 0)),
            pl.BlockSpec((TCR, nh), lambda i: (i, 0)),
            pl.BlockSpec((1, nh), lambda i: (0, 0)),
        ],
        out_shape=[
            jax.ShapeDtypeStruct((N, rw), _f32),
            jax.ShapeDtypeStruct((N, nh), _f32),
            jax.ShapeDtypeStruct((1, nh), _f32),
        ],
    )


def _make_sc_layer(rw, nh, f, acc, elu):
    fout = nh * f

    def body(hprow, ald, part, bases, gmaxp, bias, out_hbm,
             ald_tab, gmax_v, bias_v, bases_v,
             partbuf, srcbuf, scatbuf, rows, outrows, finbuf, accum,
             sem, sem2):
        c = lax.axis_index("c")
        s = lax.axis_index("s")
        w = c * NSUB + s
        iota = lax.iota(_i32, 16)
        zf = jnp.zeros((16,), _f32)

        pltpu.sync_copy(bases, bases_v)
        pltpu.sync_copy(gmaxp, gmax_v)
        pltpu.sync_copy(bias, bias_v)
        gvec = gmax_v[...]
        bvecs = [bias_v[pl.ds(i * 16, 16)] for i in range(fout // 16)]

        def zpad(g, cc):
            r16 = g * 16 + iota
            for col in range(fout + nh, acc):
                plsc.store_scatter(outrows, [r16, _full(col)], zf)
            return cc

        lax.fori_loop(0, SUP // 16, zpad, 0)

        for slot in range(2):
            b = 2 * w + slot
            node_base = b * BSZ
            nrows = jnp.minimum(BSZ, N - node_base)
            start2 = jnp.minimum(node_base, N - BSZ)
            shift = node_base - start2
            pltpu.sync_copy(ald.at[pl.ds(start2, BSZ)], ald_tab)

            def exh(dl_s, als_v, h):
                ald_v = plsc.load_gather(ald_tab, [dl_s, _full(h)])
                mp = _leaky(gvec[h] + ald_v)
                return jnp.exp(_leaky(als_v + ald_v) - mp)

            def init_chunk(k, cc):
                nb = jnp.minimum(k * 128, nrows - 128)
                pltpu.sync_copy(hprow.at[pl.ds(node_base + nb, 128)],
                                rows.at[pl.ds(0, 128)])

                def grp(g, c2):
                    r16 = g * 16 + iota
                    dl_s = nb + r16 + shift
                    exs = []
                    for h in range(nh):
                        als_v = plsc.load_gather(rows, [r16, _full(fout + h)])
                        ex = exh(dl_s, als_v, h)
                        exs.append(ex)
                        plsc.store_scatter(outrows, [r16, _full(fout + h)],
                                           ex)
                    for col in range(fout):
                        v = plsc.load_gather(rows, [r16, _full(col)])
                        plsc.store_scatter(outrows, [r16, _full(col)],
                                           v * exs[col // f])
                    return c2

                lax.fori_loop(0, 8, grp, 0)
                pltpu.sync_copy(outrows.at[pl.ds(0, 128)],
                                accum.at[pl.ds(s * BSZ + nb, 128)])
                return cc

            lax.fori_loop(0, (nrows + 127) >> 7, init_chunk, 0)

            eb = pl.multiple_of(plsc.load_gather(bases_v, [_full(b)])[0], 8)
            cnt = plsc.load_gather(bases_v, [_full(NB + b)])[0] - eb
            nsup = (cnt + SUP - 1) >> 9

            def sup_body(blk, carry):
                pltpu.sync_copy(part.at[pl.ds(eb + blk * SUP, SUP)], partbuf)

                def ug(g, cc):
                    v = plsc.load_gather(partbuf, [g * 16 + iota])
                    src = jnp.clip(v & 0x1FFFF, 0, N - 1)
                    dl = jnp.clip(jnp.right_shift(v, 17), 0, BSZ - 1)
                    plsc.store_scatter(srcbuf, [g * 16 + iota], src)
                    plsc.store_scatter(scatbuf,
                                       [_full(g >> 3), (g & 7) * 16 + iota],
                                       s * BSZ + dl)
                    return cc

                lax.fori_loop(0, SUP // 16, ug, 0)

                cps = [pltpu.async_copy(
                    hprow.at[srcbuf.at[pl.ds(j * 128, 128)]],
                    rows.at[pl.ds(j * 128, 128)], sem) for j in range(4)]
                for cp in cps:
                    cp.wait()

                def cg(g, cc):
                    r16 = g * 16 + iota
                    valid = blk * SUP + r16 < cnt
                    scat_v = plsc.load_gather(
                        scatbuf, [_full(g >> 3), (g & 7) * 16 + iota])
                    dl_s = (scat_v - s * BSZ) + shift
                    exs = []
                    for h in range(nh):
                        als_v = plsc.load_gather(rows, [r16, _full(fout + h)])
                        ex = jnp.where(valid, exh(dl_s, als_v, h), 0.0)
                        exs.append(ex)
                        plsc.store_scatter(outrows, [r16, _full(fout + h)],
                                           ex)
                    for col in range(fout):
                        v = plsc.load_gather(rows, [r16, _full(col)])
                        plsc.store_scatter(outrows, [r16, _full(col)],
                                           v * exs[col // f])
                    return cc

                lax.fori_loop(0, SUP // 16, cg, 0)

                scps = [pltpu.async_copy(
                    outrows.at[pl.ds(j * 128, 128)],
                    accum.at[scatbuf.at[j]], sem2, add=True)
                    for j in range(4)]
                for cp in scps:
                    cp.wait()
                return carry

            lax.fori_loop(0, nsup, sup_body, 0)

            def fin_chunk(k, cc):
                nb = jnp.minimum(k * 128, nrows - 128)
                pltpu.sync_copy(accum.at[pl.ds(s * BSZ + nb, 128)],
                                outrows.at[pl.ds(0, 128)])

                def grp(g, c2):
                    r16 = g * 16 + iota
                    dens = []
                    for h in range(nh):
                        d_v = plsc.load_gather(outrows,
                                               [r16, _full(fout + h)])
                        dens.append(jnp.where(d_v > 0, 1.0 / d_v, 0.0))
                    for col in range(fout):
                        v = plsc.load_gather(outrows, [r16, _full(col)])
                        o = v * dens[col // f] + bvecs[col // 16][col % 16]
                        if elu:
                            o = jnp.where(o > 0, o, jnp.exp(o) - 1.0)
                        plsc.store_scatter(finbuf, [r16, _full(col)], o)
                    return c2

                lax.fori_loop(0, 8, grp, 0)
                pltpu.sync_copy(finbuf,
                                out_hbm.at[pl.ds(node_base + nb, 128)])
                return cc

            lax.fori_loop(0, (nrows + 127) >> 7, fin_chunk, 0)

    return pl.kernel(
        body,
        out_type=jax.ShapeDtypeStruct((N, fout), _f32),
        mesh=_mesh,
        compiler_params=_sc_params,
        scratch_types=[
            pltpu.VMEM((BSZ, nh), _f32),
            pltpu.VMEM((16,), _f32),
            pltpu.VMEM((fout,), _f32),
            pltpu.VMEM((144,), _i32),
            pltpu.VMEM((SUP,), _i32),
            pltpu.VMEM((SUP,), _i32),
            pltpu.VMEM((4, 128), _i32),
            pltpu.VMEM((SUP, rw), _f32),
            pltpu.VMEM((SUP, acc), _f32),
            pltpu.VMEM((128, fout), _f32),
            pltpu.VMEM_SHARED((NSUB * BSZ, acc), _f32),
            pltpu.SemaphoreType.DMA,
            pltpu.SemaphoreType.DMA,
        ],
    )


_sc_layer01 = _make_sc_layer(rw=48, nh=4, f=8, acc=40, elu=True)
_sc_layer2 = _make_sc_layer(rw=32, nh=1, f=16, acc=24, elu=False)

_tc_prep0 = _make_tc(fin=1, fout=32, nh=4, rw=48, first=True)
_tc_prep1 = _make_tc(fin=32, fout=32, nh=4, rw=48, first=False)
_tc_prep2 = _make_tc(fin=32, fout=16, nh=1, rw=32, first=False)


def _blockdiag(a):
    nh, f = a.shape
    m = (jnp.arange(nh * f)[:, None] // f) == jnp.arange(nh)[None, :]
    return jnp.where(m, a.reshape(-1)[:, None], 0.0).astype(_f32)


def _pad16(g):
    v = g.reshape(-1)
    return jnp.pad(v, (0, 16 - v.shape[0]))


def kernel(x, edge_index, W_in, b_in, W0, asrc0, adst0, b0,
           W1, asrc1, adst1, b1, W2, asrc2, adst2, b2):
    src = jnp.pad(edge_index[0], (0, 128))
    dst = jnp.pad(edge_index[1], (0, 128))

    cnt = _p1(dst)
    part, bases = _p2(src, dst, cnt)

    hprow0, ald0, gmax0 = _tc_prep0(
        x, W_in.reshape(1, 16), b_in.reshape(1, 16), W0,
        _blockdiag(asrc0), _blockdiag(adst0))
    h1 = _sc_layer01(hprow0, ald0, part, bases, _pad16(gmax0), b0)

    hprow1, ald1, gmax1 = _tc_prep1(
        h1, W1, _blockdiag(asrc1), _blockdiag(adst1))
    h2 = _sc_layer01(hprow1, ald1, part, bases, _pad16(gmax1), b1)

    hprow2, ald2, gmax2 = _tc_prep2(
        h2, W2, _blockdiag(asrc2), _blockdiag(adst2))
    out = _sc_layer2(hprow2, ald2, part, bases, _pad16(gmax2), b2)
    return out


# speedup vs baseline: 104.4415x; 1.4801x over previous
import jax
import jax.numpy as jnp
from jax import lax
from jax.experimental import pallas as pl
from jax.experimental.pallas import tpu as pltpu
from jax.experimental.pallas import tpu_sc as plsc

N = 100000
E = 1600000
NT = 32
NSUB = 16
NB = 64
BSZ = 1568
EPT = E // NT
DIV_M = 2675
DIV_S = 17
EPAD = E + 1024
SUP = 512
TCR = 1000

_mesh = plsc.VectorSubcoreMesh(core_axis_name="c", subcore_axis_name="s")
_sc_params = pltpu.CompilerParams(needs_layout_passes=False,
                                  use_tc_tiling_on_sc=False)

_i32 = jnp.int32
_f32 = jnp.float32


def _full(v):
    return jnp.full((16,), v, _i32)


def _leaky(t):
    return jnp.where(t > 0, t, 0.2 * t)


def _bucket(d16):
    return ((d16 >> 5) * DIV_M) >> DIV_S


def _p1_body(dst_hbm, cnt_hbm, hist, dbuf):
    c = lax.axis_index("c")
    s = lax.axis_index("s")
    w = c * NSUB + s
    iota = lax.iota(_i32, 16)
    for b in range(NB):
        hist[b, :] = jnp.zeros((16,), _i32)
    ones = jnp.ones((16,), _i32)

    def chunk(k, carry):
        pltpu.sync_copy(dst_hbm.at[pl.ds(w * EPT + k * 2000, 2000)], dbuf)

        def grp(g, cc):
            d16 = plsc.load_gather(dbuf, [g * 16 + iota])
            plsc.addupdate_scatter(hist, [_bucket(d16), iota], ones)
            return cc

        return lax.fori_loop(0, 125, grp, carry)

    lax.fori_loop(0, 25, chunk, 0)
    pltpu.sync_copy(hist, cnt_hbm.at[w])


_p1 = pl.kernel(
    _p1_body,
    out_type=jax.ShapeDtypeStruct((NT, NB, 16), _i32),
    mesh=_mesh,
    compiler_params=_sc_params,
    scratch_types=[
        pltpu.VMEM((NB, 16), _i32),
        pltpu.VMEM((2000,), _i32),
    ],
)


def _p2_body(src_hbm, dst_hbm, cnt_hbm, part_hbm, bases_hbm,
             cntbuf, offtab, basesbuf, sbuf, dbuf, valbuf, posbuf, sem):
    c = lax.axis_index("c")
    s = lax.axis_index("s")
    w = c * NSUB + s
    iota = lax.iota(_i32, 16)
    pltpu.sync_copy(cnt_hbm, cntbuf)
    for b in range(9):
        basesbuf[b * 16:(b + 1) * 16] = jnp.zeros((16,), _i32)

    def off_step(g, carry):
        b = g >> 5
        wp = g & 31
        carry = jnp.where(wp == 0, (carry + 7) & ~7, carry)
        v = plsc.load_gather(cntbuf, [_full(wp), _full(b), iota])
        cs = plsc.cumsum(v)
        tot = cs[15]
        excl = carry + (cs - v)
        plsc.store_scatter(offtab, [_full(b), iota], excl,
                           mask=jnp.broadcast_to(wp == w, (16,)))
        plsc.store_scatter(basesbuf, [_full(b)], _full(carry),
                           mask=(iota == 0) & (wp == 0))
        plsc.store_scatter(basesbuf, [_full(NB + b)], _full(carry + tot),
                           mask=(iota == 0) & (wp == 31))
        return carry + tot

    lax.fori_loop(0, NB * 32, off_step, jnp.int32(0))

    @pl.when((c == 0) & (s == 0))
    def _():
        pltpu.sync_copy(basesbuf, bases_hbm)

    def do_chunk(nreal_c, nbatch):
        def grp(gi, cc):
            p16 = gi * 16 + iota
            real = p16 < nreal_c
            s16 = plsc.load_gather(sbuf, [p16])
            d16 = plsc.load_gather(dbuf, [p16])
            d16 = jnp.clip(d16, 0, N - 1)
            b = _bucket(d16)
            dl = d16 - b * BSZ
            val = jnp.where(real, s16 | jnp.left_shift(dl, 17), 0)
            pos = plsc.load_gather(offtab, [b, iota])
            plsc.store_scatter(offtab, [b, iota], pos + 1, mask=real)
            pos = jnp.where(real, pos, EPAD - 128 + (p16 & 127))
            plsc.store_scatter(valbuf, [p16], val)
            plsc.store_scatter(posbuf,
                              [_full(gi >> 3), (gi & 7) * 16 + iota], pos)
            return cc

        lax.fori_loop(0, nbatch * 8, grp, 0)
        cps = [pltpu.async_copy(valbuf.at[pl.ds(j * 128, 128)],
                                part_hbm.at[posbuf.at[j]], sem)
               for j in range(nbatch)]
        for cp in cps:
            cp.wait()

    def chunk2(ci, cc):
        cbase = w * EPT + ci * 2048
        pltpu.sync_copy(src_hbm.at[pl.ds(cbase, 2048)], sbuf)
        pltpu.sync_copy(dst_hbm.at[pl.ds(cbase, 2048)], dbuf)
        do_chunk(2048, 16)
        return cc

    lax.fori_loop(0, 24, chunk2, 0)
    tbase = w * EPT + 24 * 2048
    pltpu.sync_copy(src_hbm.at[pl.ds(tbase, 896)], sbuf.at[pl.ds(0, 896)])
    pltpu.sync_copy(dst_hbm.at[pl.ds(tbase, 896)], dbuf.at[pl.ds(0, 896)])
    do_chunk(848, 7)


_p2 = pl.kernel(
    _p2_body,
    out_type=(
        jax.ShapeDtypeStruct((EPAD,), _i32),
        jax.ShapeDtypeStruct((144,), _i32),
    ),
    mesh=_mesh,
    compiler_params=_sc_params,
    scratch_types=[
        pltpu.VMEM((NT, NB, 16), _i32),
        pltpu.VMEM((NB, 16), _i32),
        pltpu.VMEM((144,), _i32),
        pltpu.VMEM((2048,), _i32),
        pltpu.VMEM((2048,), _i32),
        pltpu.VMEM((2048,), _i32),
        pltpu.VMEM((16, 128), _i32),
        pltpu.SemaphoreType.DMA,
    ],
)


def _tc_prep_common(h, w_ref, a_ref, d_ref, hprow_ref, ald_ref, gmax_ref, i):
    hp = jnp.dot(h, w_ref[...], preferred_element_type=_f32)
    als = jnp.dot(hp, a_ref[...], preferred_element_type=_f32)
    ald = jnp.dot(hp, d_ref[...], preferred_element_type=_f32)
    fout = hp.shape[1]
    h_ = als.shape[1]
    rw = hprow_ref.shape[1]
    pad = jnp.zeros((hp.shape[0], rw - fout - h_), _f32)
    hprow_ref[...] = jnp.concatenate([hp, als, pad], axis=1)
    ald_ref[...] = ald
    bm = jnp.max(als, axis=0, keepdims=True)

    @pl.when(i == 0)
    def _():
        gmax_ref[...] = bm

    @pl.when(i > 0)
    def _():
        gmax_ref[...] = jnp.maximum(gmax_ref[...], bm)


def _tc0_body(x_ref, win_ref, bin_ref, w_ref, a_ref, d_ref,
              hprow_ref, ald_ref, gmax_ref):
    i = pl.program_id(0)
    t = x_ref[...] * win_ref[...] + bin_ref[...]
    h = jnp.where(t > 0, t, jnp.exp(t) - 1.0)
    _tc_prep_common(h, w_ref, a_ref, d_ref, hprow_ref, ald_ref, gmax_ref, i)


def _tc_body(h_ref, w_ref, a_ref, d_ref, hprow_ref, ald_ref, gmax_ref):
    i = pl.program_id(0)
    _tc_prep_common(h_ref[...], w_ref, a_ref, d_ref,
                    hprow_ref, ald_ref, gmax_ref, i)


def _make_tc(fin, fout, nh, rw, first):
    body = _tc0_body if first else _tc_body
    in_specs = [pl.BlockSpec((TCR, fin), lambda i: (i, 0))]
    kdim = 16 if first else fin
    if first:
        in_specs += [pl.BlockSpec((1, 16), lambda i: (0, 0)),
                     pl.BlockSpec((1, 16), lambda i: (0, 0))]
    in_specs += [
        pl.BlockSpec((kdim, fout), lambda i: (0, 0)),
        pl.BlockSpec((fout, nh), lambda i: (0, 0)),
        pl.BlockSpec((fout, nh), lambda i: (0, 0)),
    ]
    return pl.pallas_call(
        body,
        grid=(N // TCR,),
        in_specs=in_specs,
        out_specs=[
            pl.BlockSpec((TCR, rw), lambda i: (i, 0)),
            pl.BlockSpec((TCR, nh), lambda i: (i, 0)),
            pl.BlockSpec((1, nh), lambda i: (0, 0)),
        ],
        out_shape=[
            jax.ShapeDtypeStruct((N, rw), _f32),
            jax.ShapeDtypeStruct((N, nh), _f32),
            jax.ShapeDtypeStruct((1, nh), _f32),
        ],
    )


def _make_sc_layer(rw, nh, f, acc, elu):
    fout = nh * f

    def body(hprow, ald, part, bases, gmaxp, bias, out_hbm,
             ald_tab, gmax_v, bias_v, bases_v,
             partbuf, srcbuf, scatbuf, rows, outrows, finbuf, accum,
             sem, sem2):
        c = lax.axis_index("c")
        s = lax.axis_index("s")
        w = c * NSUB + s
        iota = lax.iota(_i32, 16)
        zf = jnp.zeros((16,), _f32)

        pltpu.sync_copy(bases, bases_v)
        pltpu.sync_copy(gmaxp, gmax_v)
        pltpu.sync_copy(bias, bias_v)
        gvec = gmax_v[...]
        bvecs = [bias_v[pl.ds(i * 16, 16)] for i in range(fout // 16)]

        def zpad(g, cc):
            r16 = g * 16 + iota
            for col in range(fout + nh, acc):
                plsc.store_scatter(outrows, [r16, _full(col)], zf)
            return cc

        lax.fori_loop(0, SUP // 16, zpad, 0)

        for slot in range(2):
            b = 2 * w + slot
            node_base = b * BSZ
            nrows = jnp.minimum(BSZ, N - node_base)
            start2 = jnp.minimum(node_base, N - BSZ)
            shift = node_base - start2
            pltpu.sync_copy(ald.at[pl.ds(start2, BSZ)], ald_tab)

            def exh(dl_s, als_v, h):
                ald_v = plsc.load_gather(ald_tab, [dl_s, _full(h)])
                mp = _leaky(gvec[h] + ald_v)
                return jnp.exp(_leaky(als_v + ald_v) - mp)

            def init_chunk(k, cc):
                nb = jnp.minimum(k * 128, nrows - 128)
                pltpu.sync_copy(hprow.at[pl.ds(node_base + nb, 128)],
                                rows.at[pl.ds(0, 128)])

                @plsc.parallel_loop(0, 8)
                def grp(g):
                    r16 = g * 16 + iota
                    dl_s = nb + r16 + shift
                    exs = []
                    for h in range(nh):
                        als_v = plsc.load_gather(rows, [r16, _full(fout + h)])
                        ex = exh(dl_s, als_v, h)
                        exs.append(ex)
                        plsc.store_scatter(outrows, [r16, _full(fout + h)],
                                           ex)
                    for col in range(fout):
                        v = plsc.load_gather(rows, [r16, _full(col)])
                        plsc.store_scatter(outrows, [r16, _full(col)],
                                           v * exs[col // f])

                pltpu.sync_copy(outrows.at[pl.ds(0, 128)],
                                accum.at[pl.ds(s * BSZ + nb, 128)])
                return cc

            lax.fori_loop(0, (nrows + 127) >> 7, init_chunk, 0)

            eb = pl.multiple_of(plsc.load_gather(bases_v, [_full(b)])[0], 8)
            cnt = plsc.load_gather(bases_v, [_full(NB + b)])[0] - eb
            nsup = (cnt + SUP - 1) >> 9

            def sup_body(blk, carry):
                pltpu.sync_copy(part.at[pl.ds(eb + blk * SUP, SUP)], partbuf)

                @plsc.parallel_loop(0, SUP // 16, unroll=2)
                def ug(g):
                    v = plsc.load_gather(partbuf, [g * 16 + iota])
                    src = jnp.clip(v & 0x1FFFF, 0, N - 1)
                    dl = jnp.clip(jnp.right_shift(v, 17), 0, BSZ - 1)
                    plsc.store_scatter(srcbuf, [g * 16 + iota], src)
                    plsc.store_scatter(scatbuf,
                                       [_full(g >> 3), (g & 7) * 16 + iota],
                                       s * BSZ + dl)

                cps = [pltpu.async_copy(
                    hprow.at[srcbuf.at[pl.ds(j * 128, 128)]],
                    rows.at[pl.ds(j * 128, 128)], sem) for j in range(4)]
                for cp in cps:
                    cp.wait()

                def cg(g):
                    r16 = g * 16 + iota
                    valid = blk * SUP + r16 < cnt
                    scat_v = plsc.load_gather(
                        scatbuf, [_full(g >> 3), (g & 7) * 16 + iota])
                    dl_s = (scat_v - s * BSZ) + shift
                    exs = []
                    for h in range(nh):
                        als_v = plsc.load_gather(rows, [r16, _full(fout + h)])
                        ex = jnp.where(valid, exh(dl_s, als_v, h), 0.0)
                        exs.append(ex)
                        plsc.store_scatter(outrows, [r16, _full(fout + h)],
                                           ex)
                    for col in range(fout):
                        v = plsc.load_gather(rows, [r16, _full(col)])
                        plsc.store_scatter(outrows, [r16, _full(col)],
                                           v * exs[col // f])

                plsc.parallel_loop(0, SUP // 16)(cg)

                scps = [pltpu.async_copy(
                    outrows.at[pl.ds(j * 128, 128)],
                    accum.at[scatbuf.at[j]], sem2, add=True)
                    for j in range(4)]
                for cp in scps:
                    cp.wait()
                return carry

            lax.fori_loop(0, nsup, sup_body, 0)

            def fin_chunk(k, cc):
                nb = jnp.minimum(k * 128, nrows - 128)
                pltpu.sync_copy(accum.at[pl.ds(s * BSZ + nb, 128)],
                                outrows.at[pl.ds(0, 128)])

                @plsc.parallel_loop(0, 8)
                def grp(g):
                    r16 = g * 16 + iota
                    dens = []
                    for h in range(nh):
                        d_v = plsc.load_gather(outrows,
                                               [r16, _full(fout + h)])
                        dens.append(jnp.where(d_v > 0, 1.0 / d_v, 0.0))
                    for col in range(fout):
                        v = plsc.load_gather(outrows, [r16, _full(col)])
                        o = v * dens[col // f] + bvecs[col // 16][col % 16]
                        if elu:
                            o = jnp.where(o > 0, o, jnp.exp(o) - 1.0)
                        plsc.store_scatter(finbuf, [r16, _full(col)], o)

                pltpu.sync_copy(finbuf,
                                out_hbm.at[pl.ds(node_base + nb, 128)])
                return cc

            lax.fori_loop(0, (nrows + 127) >> 7, fin_chunk, 0)

    return pl.kernel(
        body,
        out_type=jax.ShapeDtypeStruct((N, fout), _f32),
        mesh=_mesh,
        compiler_params=_sc_params,
        scratch_types=[
            pltpu.VMEM((BSZ, nh), _f32),
            pltpu.VMEM((16,), _f32),
            pltpu.VMEM((fout,), _f32),
            pltpu.VMEM((144,), _i32),
            pltpu.VMEM((SUP,), _i32),
            pltpu.VMEM((SUP,), _i32),
            pltpu.VMEM((4, 128), _i32),
            pltpu.VMEM((SUP, rw), _f32),
            pltpu.VMEM((SUP, acc), _f32),
            pltpu.VMEM((128, fout), _f32),
            pltpu.VMEM_SHARED((NSUB * BSZ, acc), _f32),
            pltpu.SemaphoreType.DMA,
            pltpu.SemaphoreType.DMA,
        ],
    )


_sc_layer01 = _make_sc_layer(rw=48, nh=4, f=8, acc=40, elu=True)
_sc_layer2 = _make_sc_layer(rw=32, nh=1, f=16, acc=24, elu=False)

_tc_prep0 = _make_tc(fin=1, fout=32, nh=4, rw=48, first=True)
_tc_prep1 = _make_tc(fin=32, fout=32, nh=4, rw=48, first=False)
_tc_prep2 = _make_tc(fin=32, fout=16, nh=1, rw=32, first=False)


def _blockdiag(a):
    nh, f = a.shape
    m = (jnp.arange(nh * f)[:, None] // f) == jnp.arange(nh)[None, :]
    return jnp.where(m, a.reshape(-1)[:, None], 0.0).astype(_f32)


def _pad16(g):
    v = g.reshape(-1)
    return jnp.pad(v, (0, 16 - v.shape[0]))


def kernel(x, edge_index, W_in, b_in, W0, asrc0, adst0, b0,
           W1, asrc1, adst1, b1, W2, asrc2, adst2, b2):
    src = jnp.pad(edge_index[0], (0, 128))
    dst = jnp.pad(edge_index[1], (0, 128))

    cnt = _p1(dst)
    part, bases = _p2(src, dst, cnt)

    hprow0, ald0, gmax0 = _tc_prep0(
        x, W_in.reshape(1, 16), b_in.reshape(1, 16), W0,
        _blockdiag(asrc0), _blockdiag(adst0))
    h1 = _sc_layer01(hprow0, ald0, part, bases, _pad16(gmax0), b0)

    hprow1, ald1, gmax1 = _tc_prep1(
        h1, W1, _blockdiag(asrc1), _blockdiag(adst1))
    h2 = _sc_layer01(hprow1, ald1, part, bases, _pad16(gmax1), b1)

    hprow2, ald2, gmax2 = _tc_prep2(
        h2, W2, _blockdiag(asrc2), _blockdiag(adst2))
    out = _sc_layer2(hprow2, ald2, part, bases, _pad16(gmax2), b2)
    return out


# speedup vs baseline: 104.5616x; 1.0011x over previous
import jax
import jax.numpy as jnp
from jax import lax
from jax.experimental import pallas as pl
from jax.experimental.pallas import tpu as pltpu
from jax.experimental.pallas import tpu_sc as plsc

N = 100000
E = 1600000
NT = 32
NSUB = 16
NB = 64
BSZ = 1568
EPT = E // NT
DIV_M = 2675
DIV_S = 17
EPAD = E + 1024
SUP = 512
TCR = 1000

_mesh = plsc.VectorSubcoreMesh(core_axis_name="c", subcore_axis_name="s")
_sc_params = pltpu.CompilerParams(needs_layout_passes=False,
                                  use_tc_tiling_on_sc=False)

_i32 = jnp.int32
_f32 = jnp.float32


def _full(v):
    return jnp.full((16,), v, _i32)


def _leaky(t):
    return jnp.where(t > 0, t, 0.2 * t)


def _bucket(d16):
    return ((d16 >> 5) * DIV_M) >> DIV_S


def _p1_body(dst_hbm, cnt_hbm, hist, dbuf):
    c = lax.axis_index("c")
    s = lax.axis_index("s")
    w = c * NSUB + s
    iota = lax.iota(_i32, 16)
    for b in range(NB):
        hist[b, :] = jnp.zeros((16,), _i32)
    ones = jnp.ones((16,), _i32)

    def chunk(k, carry):
        pltpu.sync_copy(dst_hbm.at[pl.ds(w * EPT + k * 2000, 2000)], dbuf)

        def grp(g, cc):
            d16 = plsc.load_gather(dbuf, [g * 16 + iota])
            plsc.addupdate_scatter(hist, [_bucket(d16), iota], ones)
            return cc

        return lax.fori_loop(0, 125, grp, carry)

    lax.fori_loop(0, 25, chunk, 0)
    pltpu.sync_copy(hist, cnt_hbm.at[w])


_p1 = pl.kernel(
    _p1_body,
    out_type=jax.ShapeDtypeStruct((NT, NB, 16), _i32),
    mesh=_mesh,
    compiler_params=_sc_params,
    scratch_types=[
        pltpu.VMEM((NB, 16), _i32),
        pltpu.VMEM((2000,), _i32),
    ],
)


def _p2_body(src_hbm, dst_hbm, cnt_hbm, part_hbm, bases_hbm,
             cntbuf, offtab, basesbuf, sbuf, dbuf, valbuf, posbuf, sem):
    c = lax.axis_index("c")
    s = lax.axis_index("s")
    w = c * NSUB + s
    iota = lax.iota(_i32, 16)
    pltpu.sync_copy(cnt_hbm, cntbuf)
    for b in range(9):
        basesbuf[b * 16:(b + 1) * 16] = jnp.zeros((16,), _i32)

    def off_step(g, carry):
        b = g >> 5
        wp = g & 31
        carry = jnp.where(wp == 0, (carry + 7) & ~7, carry)
        v = plsc.load_gather(cntbuf, [_full(wp), _full(b), iota])
        cs = plsc.cumsum(v)
        tot = cs[15]
        excl = carry + (cs - v)
        plsc.store_scatter(offtab, [_full(b), iota], excl,
                           mask=jnp.broadcast_to(wp == w, (16,)))
        plsc.store_scatter(basesbuf, [_full(b)], _full(carry),
                           mask=(iota == 0) & (wp == 0))
        plsc.store_scatter(basesbuf, [_full(NB + b)], _full(carry + tot),
                           mask=(iota == 0) & (wp == 31))
        return carry + tot

    lax.fori_loop(0, NB * 32, off_step, jnp.int32(0))

    @pl.when((c == 0) & (s == 0))
    def _():
        pltpu.sync_copy(basesbuf, bases_hbm)

    def do_grp(nreal_c, pb):
        def grp(gi, cc):
            p16 = gi * 16 + iota
            real = p16 < nreal_c
            s16 = plsc.load_gather(sbuf, [p16])
            d16 = plsc.load_gather(dbuf, [p16])
            d16 = jnp.clip(d16, 0, N - 1)
            b = _bucket(d16)
            dl = d16 - b * BSZ
            val = jnp.where(real, s16 | jnp.left_shift(dl, 17), 0)
            pos = plsc.load_gather(offtab, [b, iota])
            plsc.store_scatter(offtab, [b, iota], pos + 1, mask=real)
            pos = jnp.where(real, pos, EPAD - 128 + (p16 & 127))
            plsc.store_scatter(valbuf, [pb * 2048 + p16], val)
            plsc.store_scatter(posbuf,
                              [_full(pb * 16 + (gi >> 3)),
                               (gi & 7) * 16 + iota], pos)
            return cc

        lax.fori_loop(0, (nreal_c + 127) // 128 * 8, grp, 0)

    def fire(pb, nbatch):
        return [pltpu.async_copy(
            valbuf.at[pl.ds(pb * 2048 + j * 128, 128)],
            part_hbm.at[posbuf.at[pb * 16 + j]], sem)
            for j in range(nbatch)]

    def drain(pb, nbatch):
        for j in range(nbatch):
            pltpu.make_async_copy(
                valbuf.at[pl.ds(pb * 2048 + j * 128, 128)],
                part_hbm.at[posbuf.at[pb * 16 + j]], sem).wait()

    def chunk2(ci, cc):
        cbase = w * EPT + ci * 2048
        pltpu.sync_copy(src_hbm.at[pl.ds(cbase, 2048)], sbuf)
        pltpu.sync_copy(dst_hbm.at[pl.ds(cbase, 2048)], dbuf)
        do_grp(2048, ci & 1)

        @pl.when(ci > 0)
        def _():
            drain(1 - (ci & 1), 16)

        fire(ci & 1, 16)
        return cc

    lax.fori_loop(0, 24, chunk2, 0)
    drain(1, 16)
    tbase = w * EPT + 24 * 2048
    pltpu.sync_copy(src_hbm.at[pl.ds(tbase, 896)], sbuf.at[pl.ds(0, 896)])
    pltpu.sync_copy(dst_hbm.at[pl.ds(tbase, 896)], dbuf.at[pl.ds(0, 896)])
    do_grp(848, 0)
    fire(0, 7)
    drain(0, 7)


_p2 = pl.kernel(
    _p2_body,
    out_type=(
        jax.ShapeDtypeStruct((EPAD,), _i32),
        jax.ShapeDtypeStruct((144,), _i32),
    ),
    mesh=_mesh,
    compiler_params=_sc_params,
    scratch_types=[
        pltpu.VMEM((NT, NB, 16), _i32),
        pltpu.VMEM((NB, 16), _i32),
        pltpu.VMEM((144,), _i32),
        pltpu.VMEM((2048,), _i32),
        pltpu.VMEM((2048,), _i32),
        pltpu.VMEM((4096,), _i32),
        pltpu.VMEM((32, 128), _i32),
        pltpu.SemaphoreType.DMA,
    ],
)


def _tc_prep_common(h, w_ref, a_ref, d_ref, hprow_ref, ald_ref, gmax_ref, i):
    hp = jnp.dot(h, w_ref[...], preferred_element_type=_f32)
    als = jnp.dot(hp, a_ref[...], preferred_element_type=_f32)
    ald = jnp.dot(hp, d_ref[...], preferred_element_type=_f32)
    fout = hp.shape[1]
    h_ = als.shape[1]
    rw = hprow_ref.shape[1]
    pad = jnp.zeros((hp.shape[0], rw - fout - h_), _f32)
    hprow_ref[...] = jnp.concatenate([hp, als, pad], axis=1)
    ald_ref[...] = ald
    bm = jnp.max(als, axis=0, keepdims=True)

    @pl.when(i == 0)
    def _():
        gmax_ref[...] = bm

    @pl.when(i > 0)
    def _():
        gmax_ref[...] = jnp.maximum(gmax_ref[...], bm)


def _tc0_body(x_ref, win_ref, bin_ref, w_ref, a_ref, d_ref,
              hprow_ref, ald_ref, gmax_ref):
    i = pl.program_id(0)
    t = x_ref[...] * win_ref[...] + bin_ref[...]
    h = jnp.where(t > 0, t, jnp.exp(t) - 1.0)
    _tc_prep_common(h, w_ref, a_ref, d_ref, hprow_ref, ald_ref, gmax_ref, i)


def _tc_body(h_ref, w_ref, a_ref, d_ref, hprow_ref, ald_ref, gmax_ref):
    i = pl.program_id(0)
    _tc_prep_common(h_ref[...], w_ref, a_ref, d_ref,
                    hprow_ref, ald_ref, gmax_ref, i)


def _make_tc(fin, fout, nh, rw, first):
    body = _tc0_body if first else _tc_body
    in_specs = [pl.BlockSpec((TCR, fin), lambda i: (i, 0))]
    kdim = 16 if first else fin
    if first:
        in_specs += [pl.BlockSpec((1, 16), lambda i: (0, 0)),
                     pl.BlockSpec((1, 16), lambda i: (0, 0))]
    in_specs += [
        pl.BlockSpec((kdim, fout), lambda i: (0, 0)),
        pl.BlockSpec((fout, nh), lambda i: (0, 0)),
        pl.BlockSpec((fout, nh), lambda i: (0, 0)),
    ]
    return pl.pallas_call(
        body,
        grid=(N // TCR,),
        in_specs=in_specs,
        out_specs=[
            pl.BlockSpec((TCR, rw), lambda i: (i, 0)),
            pl.BlockSpec((TCR, nh), lambda i: (i, 0)),
            pl.BlockSpec((1, nh), lambda i: (0, 0)),
        ],
        out_shape=[
            jax.ShapeDtypeStruct((N, rw), _f32),
            jax.ShapeDtypeStruct((N, nh), _f32),
            jax.ShapeDtypeStruct((1, nh), _f32),
        ],
    )


def _make_sc_layer(rw, nh, f, acc, elu):
    fout = nh * f

    def body(hprow, ald, part, bases, gmaxp, bias, out_hbm,
             ald_tab, gmax_v, bias_v, bases_v,
             partbuf, srcbuf, scatbuf, rows, outrows, finbuf, accum,
             sem, sem2):
        c = lax.axis_index("c")
        s = lax.axis_index("s")
        w = c * NSUB + s
        iota = lax.iota(_i32, 16)
        zf = jnp.zeros((16,), _f32)

        pltpu.sync_copy(bases, bases_v)
        pltpu.sync_copy(gmaxp, gmax_v)
        pltpu.sync_copy(bias, bias_v)
        gvec = gmax_v[...]
        bvecs = [bias_v[pl.ds(i * 16, 16)] for i in range(fout // 16)]

        def zpad(g, cc):
            r16 = g * 16 + iota
            for col in range(fout + nh, acc):
                plsc.store_scatter(outrows, [r16, _full(col)], zf)
            return cc

        lax.fori_loop(0, SUP // 16, zpad, 0)

        for slot in range(2):
            b = 2 * w + slot
            node_base = b * BSZ
            nrows = jnp.minimum(BSZ, N - node_base)
            start2 = jnp.minimum(node_base, N - BSZ)
            shift = node_base - start2
            pltpu.sync_copy(ald.at[pl.ds(start2, BSZ)], ald_tab)

            def exh(dl_s, als_v, h):
                ald_v = plsc.load_gather(ald_tab, [dl_s, _full(h)])
                mp = _leaky(gvec[h] + ald_v)
                return jnp.exp(_leaky(als_v + ald_v) - mp)

            def init_chunk(k, cc):
                nb = jnp.minimum(k * 128, nrows - 128)
                pltpu.sync_copy(hprow.at[pl.ds(node_base + nb, 128)],
                                rows.at[pl.ds(0, 128)])

                @plsc.parallel_loop(0, 8)
                def grp(g):
                    r16 = g * 16 + iota
                    dl_s = nb + r16 + shift
                    exs = []
                    for h in range(nh):
                        als_v = plsc.load_gather(rows, [r16, _full(fout + h)])
                        ex = exh(dl_s, als_v, h)
                        exs.append(ex)
                        plsc.store_scatter(outrows, [r16, _full(fout + h)],
                                           ex)
                    for col in range(fout):
                        v = plsc.load_gather(rows, [r16, _full(col)])
                        plsc.store_scatter(outrows, [r16, _full(col)],
                                           v * exs[col // f])

                pltpu.sync_copy(outrows.at[pl.ds(0, 128)],
                                accum.at[pl.ds(s * BSZ + nb, 128)])
                return cc

            lax.fori_loop(0, (nrows + 127) >> 7, init_chunk, 0)

            eb = pl.multiple_of(plsc.load_gather(bases_v, [_full(b)])[0], 8)
            cnt = plsc.load_gather(bases_v, [_full(NB + b)])[0] - eb
            nsup = (cnt + SUP - 1) >> 9

            def sup_body(blk, carry):
                pltpu.sync_copy(part.at[pl.ds(eb + blk * SUP, SUP)], partbuf)

                @plsc.parallel_loop(0, SUP // 16, unroll=2)
                def ug(g):
                    v = plsc.load_gather(partbuf, [g * 16 + iota])
                    src = jnp.clip(v & 0x1FFFF, 0, N - 1)
                    dl = jnp.clip(jnp.right_shift(v, 17), 0, BSZ - 1)
                    plsc.store_scatter(srcbuf, [g * 16 + iota], src)
                    plsc.store_scatter(scatbuf,
                                       [_full(g >> 3), (g & 7) * 16 + iota],
                                       s * BSZ + dl)

                cps = [pltpu.async_copy(
                    hprow.at[srcbuf.at[pl.ds(j * 128, 128)]],
                    rows.at[pl.ds(j * 128, 128)], sem) for j in range(4)]
                for cp in cps:
                    cp.wait()

                def cg(g):
                    r16 = g * 16 + iota
                    valid = blk * SUP + r16 < cnt
                    scat_v = plsc.load_gather(
                        scatbuf, [_full(g >> 3), (g & 7) * 16 + iota])
                    dl_s = (scat_v - s * BSZ) + shift
                    exs = []
                    for h in range(nh):
                        als_v = plsc.load_gather(rows, [r16, _full(fout + h)])
                        ex = jnp.where(valid, exh(dl_s, als_v, h), 0.0)
                        exs.append(ex)
                        plsc.store_scatter(outrows, [r16, _full(fout + h)],
                                           ex)
                    for col in range(fout):
                        v = plsc.load_gather(rows, [r16, _full(col)])
                        plsc.store_scatter(outrows, [r16, _full(col)],
                                           v * exs[col // f])

                plsc.parallel_loop(0, SUP // 16)(cg)

                scps = [pltpu.async_copy(
                    outrows.at[pl.ds(j * 128, 128)],
                    accum.at[scatbuf.at[j]], sem2, add=True)
                    for j in range(4)]
                for cp in scps:
                    cp.wait()
                return carry

            lax.fori_loop(0, nsup, sup_body, 0)

            def fin_chunk(k, cc):
                nb = jnp.minimum(k * 128, nrows - 128)
                pltpu.sync_copy(accum.at[pl.ds(s * BSZ + nb, 128)],
                                outrows.at[pl.ds(0, 128)])

                @plsc.parallel_loop(0, 8)
                def grp(g):
                    r16 = g * 16 + iota
                    dens = []
                    for h in range(nh):
                        d_v = plsc.load_gather(outrows,
                                               [r16, _full(fout + h)])
                        dens.append(jnp.where(d_v > 0, 1.0 / d_v, 0.0))
                    for col in range(fout):
                        v = plsc.load_gather(outrows, [r16, _full(col)])
                        o = v * dens[col // f] + bvecs[col // 16][col % 16]
                        if elu:
                            o = jnp.where(o > 0, o, jnp.exp(o) - 1.0)
                        plsc.store_scatter(finbuf, [r16, _full(col)], o)

                pltpu.sync_copy(finbuf,
                                out_hbm.at[pl.ds(node_base + nb, 128)])
                return cc

            lax.fori_loop(0, (nrows + 127) >> 7, fin_chunk, 0)

    return pl.kernel(
        body,
        out_type=jax.ShapeDtypeStruct((N, fout), _f32),
        mesh=_mesh,
        compiler_params=_sc_params,
        scratch_types=[
            pltpu.VMEM((BSZ, nh), _f32),
            pltpu.VMEM((16,), _f32),
            pltpu.VMEM((fout,), _f32),
            pltpu.VMEM((144,), _i32),
            pltpu.VMEM((SUP,), _i32),
            pltpu.VMEM((SUP,), _i32),
            pltpu.VMEM((4, 128), _i32),
            pltpu.VMEM((SUP, rw), _f32),
            pltpu.VMEM((SUP, acc), _f32),
            pltpu.VMEM((128, fout), _f32),
            pltpu.VMEM_SHARED((NSUB * BSZ, acc), _f32),
            pltpu.SemaphoreType.DMA,
            pltpu.SemaphoreType.DMA,
        ],
    )


_sc_layer01 = _make_sc_layer(rw=48, nh=4, f=8, acc=40, elu=True)
_sc_layer2 = _make_sc_layer(rw=32, nh=1, f=16, acc=24, elu=False)

_tc_prep0 = _make_tc(fin=1, fout=32, nh=4, rw=48, first=True)
_tc_prep1 = _make_tc(fin=32, fout=32, nh=4, rw=48, first=False)
_tc_prep2 = _make_tc(fin=32, fout=16, nh=1, rw=32, first=False)


def _blockdiag(a):
    nh, f = a.shape
    m = (jnp.arange(nh * f)[:, None] // f) == jnp.arange(nh)[None, :]
    return jnp.where(m, a.reshape(-1)[:, None], 0.0).astype(_f32)


def _pad16(g):
    v = g.reshape(-1)
    return jnp.pad(v, (0, 16 - v.shape[0]))


def kernel(x, edge_index, W_in, b_in, W0, asrc0, adst0, b0,
           W1, asrc1, adst1, b1, W2, asrc2, adst2, b2):
    src = jnp.pad(edge_index[0], (0, 128))
    dst = jnp.pad(edge_index[1], (0, 128))

    cnt = _p1(dst)
    part, bases = _p2(src, dst, cnt)

    hprow0, ald0, gmax0 = _tc_prep0(
        x, W_in.reshape(1, 16), b_in.reshape(1, 16), W0,
        _blockdiag(asrc0), _blockdiag(adst0))
    h1 = _sc_layer01(hprow0, ald0, part, bases, _pad16(gmax0), b0)

    hprow1, ald1, gmax1 = _tc_prep1(
        h1, W1, _blockdiag(asrc1), _blockdiag(adst1))
    h2 = _sc_layer01(hprow1, ald1, part, bases, _pad16(gmax1), b1)

    hprow2, ald2, gmax2 = _tc_prep2(
        h2, W2, _blockdiag(asrc2), _blockdiag(adst2))
    out = _sc_layer2(hprow2, ald2, part, bases, _pad16(gmax2), b2)
    return out


# speedup vs baseline: 104.7065x; 1.0014x over previous
import jax
import jax.numpy as jnp
from jax import lax
from jax.experimental import pallas as pl
from jax.experimental.pallas import tpu as pltpu
from jax.experimental.pallas import tpu_sc as plsc

N = 100000
E = 1600000
NT = 32
NSUB = 16
NB = 64
BSZ = 1568
EPT = E // NT
DIV_M = 2675
DIV_S = 17
EPAD = E + 1024
SUP = 512
TCR = 1000

_mesh = plsc.VectorSubcoreMesh(core_axis_name="c", subcore_axis_name="s")
_sc_params = pltpu.CompilerParams(needs_layout_passes=False,
                                  use_tc_tiling_on_sc=False)

_i32 = jnp.int32
_f32 = jnp.float32


def _full(v):
    return jnp.full((16,), v, _i32)


def _leaky(t):
    return jnp.where(t > 0, t, 0.2 * t)


def _bucket(d16):
    return ((d16 >> 5) * DIV_M) >> DIV_S


def _p1_body(dst_hbm, cnt_hbm, hist, dbuf):
    c = lax.axis_index("c")
    s = lax.axis_index("s")
    w = c * NSUB + s
    iota = lax.iota(_i32, 16)
    for b in range(NB):
        hist[b, :] = jnp.zeros((16,), _i32)
    ones = jnp.ones((16,), _i32)

    def chunk(k, carry):
        pltpu.sync_copy(dst_hbm.at[pl.ds(w * EPT + k * 2000, 2000)], dbuf)

        def grp(g, cc):
            d16 = plsc.load_gather(dbuf, [g * 16 + iota])
            plsc.addupdate_scatter(hist, [_bucket(d16), iota], ones)
            return cc

        return lax.fori_loop(0, 125, grp, carry)

    lax.fori_loop(0, 25, chunk, 0)
    pltpu.sync_copy(hist, cnt_hbm.at[w])


_p1 = pl.kernel(
    _p1_body,
    out_type=jax.ShapeDtypeStruct((NT, NB, 16), _i32),
    mesh=_mesh,
    compiler_params=_sc_params,
    scratch_types=[
        pltpu.VMEM((NB, 16), _i32),
        pltpu.VMEM((2000,), _i32),
    ],
)


def _p2_body(src_hbm, dst_hbm, cnt_hbm, part_hbm, bases_hbm,
             cntbuf, offtab, basesbuf, sbuf, dbuf, valbuf, posbuf, bbuf, sem):
    c = lax.axis_index("c")
    s = lax.axis_index("s")
    w = c * NSUB + s
    iota = lax.iota(_i32, 16)
    pltpu.sync_copy(cnt_hbm, cntbuf)
    for b in range(9):
        basesbuf[b * 16:(b + 1) * 16] = jnp.zeros((16,), _i32)

    def off_step(g, carry):
        b = g >> 5
        wp = g & 31
        carry = jnp.where(wp == 0, (carry + 7) & ~7, carry)
        v = plsc.load_gather(cntbuf, [_full(wp), _full(b), iota])
        cs = plsc.cumsum(v)
        tot = cs[15]
        excl = carry + (cs - v)
        plsc.store_scatter(offtab, [_full(b), iota], excl,
                           mask=jnp.broadcast_to(wp == w, (16,)))
        plsc.store_scatter(basesbuf, [_full(b)], _full(carry),
                           mask=(iota == 0) & (wp == 0))
        plsc.store_scatter(basesbuf, [_full(NB + b)], _full(carry + tot),
                           mask=(iota == 0) & (wp == 31))
        return carry + tot

    lax.fori_loop(0, NB * 32, off_step, jnp.int32(0))

    @pl.when((c == 0) & (s == 0))
    def _():
        pltpu.sync_copy(basesbuf, bases_hbm)

    def do_grp(nreal_c, pb):
        ngrp = (nreal_c + 127) // 128 * 8

        @plsc.parallel_loop(0, ngrp, unroll=2)
        def pa(gi):
            p16 = gi * 16 + iota
            real = p16 < nreal_c
            s16 = plsc.load_gather(sbuf, [p16])
            d16 = plsc.load_gather(dbuf, [p16])
            d16 = jnp.clip(d16, 0, N - 1)
            b = _bucket(d16)
            dl = d16 - b * BSZ
            val = jnp.where(real, s16 | jnp.left_shift(dl, 17), 0)
            plsc.store_scatter(bbuf, [p16], b)
            plsc.store_scatter(valbuf, [pb * 2048 + p16], val)

        def pbl(gi, cc):
            p16 = gi * 16 + iota
            real = p16 < nreal_c
            b = plsc.load_gather(bbuf, [p16])
            pos = plsc.load_gather(offtab, [b, iota])
            plsc.store_scatter(offtab, [b, iota], pos + 1, mask=real)
            pos = jnp.where(real, pos, EPAD - 128 + (p16 & 127))
            plsc.store_scatter(posbuf,
                              [_full(pb * 16 + (gi >> 3)),
                               (gi & 7) * 16 + iota], pos)
            return cc

        lax.fori_loop(0, ngrp, pbl, 0)

    def fire(pb, nbatch):
        return [pltpu.async_copy(
            valbuf.at[pl.ds(pb * 2048 + j * 128, 128)],
            part_hbm.at[posbuf.at[pb * 16 + j]], sem)
            for j in range(nbatch)]

    def drain(pb, nbatch):
        for j in range(nbatch):
            pltpu.make_async_copy(
                valbuf.at[pl.ds(pb * 2048 + j * 128, 128)],
                part_hbm.at[posbuf.at[pb * 16 + j]], sem).wait()

    def chunk2(ci, cc):
        cbase = w * EPT + ci * 2048
        pltpu.sync_copy(src_hbm.at[pl.ds(cbase, 2048)], sbuf)
        pltpu.sync_copy(dst_hbm.at[pl.ds(cbase, 2048)], dbuf)
        do_grp(2048, ci & 1)

        @pl.when(ci > 0)
        def _():
            drain(1 - (ci & 1), 16)

        fire(ci & 1, 16)
        return cc

    lax.fori_loop(0, 24, chunk2, 0)
    drain(1, 16)
    tbase = w * EPT + 24 * 2048
    pltpu.sync_copy(src_hbm.at[pl.ds(tbase, 896)], sbuf.at[pl.ds(0, 896)])
    pltpu.sync_copy(dst_hbm.at[pl.ds(tbase, 896)], dbuf.at[pl.ds(0, 896)])
    do_grp(848, 0)
    fire(0, 7)
    drain(0, 7)


_p2 = pl.kernel(
    _p2_body,
    out_type=(
        jax.ShapeDtypeStruct((EPAD,), _i32),
        jax.ShapeDtypeStruct((144,), _i32),
    ),
    mesh=_mesh,
    compiler_params=_sc_params,
    scratch_types=[
        pltpu.VMEM((NT, NB, 16), _i32),
        pltpu.VMEM((NB, 16), _i32),
        pltpu.VMEM((144,), _i32),
        pltpu.VMEM((2048,), _i32),
        pltpu.VMEM((2048,), _i32),
        pltpu.VMEM((4096,), _i32),
        pltpu.VMEM((32, 128), _i32),
        pltpu.VMEM((2048,), _i32),
        pltpu.SemaphoreType.DMA,
    ],
)


def _tc_prep_common(h, w_ref, a_ref, d_ref, hprow_ref, ald_ref, gmax_ref, i):
    hp = jnp.dot(h, w_ref[...], preferred_element_type=_f32)
    als = jnp.dot(hp, a_ref[...], preferred_element_type=_f32)
    ald = jnp.dot(hp, d_ref[...], preferred_element_type=_f32)
    fout = hp.shape[1]
    h_ = als.shape[1]
    rw = hprow_ref.shape[1]
    pad = jnp.zeros((hp.shape[0], rw - fout - h_), _f32)
    hprow_ref[...] = jnp.concatenate([hp, als, pad], axis=1)
    ald_ref[...] = ald
    bm = jnp.max(als, axis=0, keepdims=True)

    @pl.when(i == 0)
    def _():
        gmax_ref[...] = bm

    @pl.when(i > 0)
    def _():
        gmax_ref[...] = jnp.maximum(gmax_ref[...], bm)


def _tc0_body(x_ref, win_ref, bin_ref, w_ref, a_ref, d_ref,
              hprow_ref, ald_ref, gmax_ref):
    i = pl.program_id(0)
    t = x_ref[...] * win_ref[...] + bin_ref[...]
    h = jnp.where(t > 0, t, jnp.exp(t) - 1.0)
    _tc_prep_common(h, w_ref, a_ref, d_ref, hprow_ref, ald_ref, gmax_ref, i)


def _tc_body(h_ref, w_ref, a_ref, d_ref, hprow_ref, ald_ref, gmax_ref):
    i = pl.program_id(0)
    _tc_prep_common(h_ref[...], w_ref, a_ref, d_ref,
                    hprow_ref, ald_ref, gmax_ref, i)


def _make_tc(fin, fout, nh, rw, first):
    body = _tc0_body if first else _tc_body
    in_specs = [pl.BlockSpec((TCR, fin), lambda i: (i, 0))]
    kdim = 16 if first else fin
    if first:
        in_specs += [pl.BlockSpec((1, 16), lambda i: (0, 0)),
                     pl.BlockSpec((1, 16), lambda i: (0, 0))]
    in_specs += [
        pl.BlockSpec((kdim, fout), lambda i: (0, 0)),
        pl.BlockSpec((fout, nh), lambda i: (0, 0)),
        pl.BlockSpec((fout, nh), lambda i: (0, 0)),
    ]
    return pl.pallas_call(
        body,
        grid=(N // TCR,),
        in_specs=in_specs,
        out_specs=[
            pl.BlockSpec((TCR, rw), lambda i: (i, 0)),
            pl.BlockSpec((TCR, nh), lambda i: (i, 0)),
            pl.BlockSpec((1, nh), lambda i: (0, 0)),
        ],
        out_shape=[
            jax.ShapeDtypeStruct((N, rw), _f32),
            jax.ShapeDtypeStruct((N, nh), _f32),
            jax.ShapeDtypeStruct((1, nh), _f32),
        ],
    )


def _make_sc_layer(rw, nh, f, acc, elu):
    fout = nh * f

    def body(hprow, ald, part, bases, gmaxp, bias, out_hbm,
             ald_tab, gmax_v, bias_v, bases_v,
             partbuf, srcbuf, scatbuf, rows, outrows, finbuf, accum,
             sem, sem2):
        c = lax.axis_index("c")
        s = lax.axis_index("s")
        w = c * NSUB + s
        iota = lax.iota(_i32, 16)
        zf = jnp.zeros((16,), _f32)

        pltpu.sync_copy(bases, bases_v)
        pltpu.sync_copy(gmaxp, gmax_v)
        pltpu.sync_copy(bias, bias_v)
        gvec = gmax_v[...]
        bvecs = [bias_v[pl.ds(i * 16, 16)] for i in range(fout // 16)]

        def zpad(g, cc):
            r16 = g * 16 + iota
            for col in range(fout + nh, acc):
                plsc.store_scatter(outrows, [r16, _full(col)], zf)
            return cc

        lax.fori_loop(0, SUP // 16, zpad, 0)

        for slot in range(2):
            b = 2 * w + slot
            node_base = b * BSZ
            nrows = jnp.minimum(BSZ, N - node_base)
            start2 = jnp.minimum(node_base, N - BSZ)
            shift = node_base - start2
            pltpu.sync_copy(ald.at[pl.ds(start2, BSZ)], ald_tab)

            def exh(dl_s, als_v, h):
                ald_v = plsc.load_gather(ald_tab, [dl_s, _full(h)])
                mp = _leaky(gvec[h] + ald_v)
                return jnp.exp(_leaky(als_v + ald_v) - mp)

            def init_chunk(k, cc):
                nb = jnp.minimum(k * 128, nrows - 128)
                pltpu.sync_copy(hprow.at[pl.ds(node_base + nb, 128)],
                                rows.at[pl.ds(0, 128)])

                @plsc.parallel_loop(0, 8)
                def grp(g):
                    r16 = g * 16 + iota
                    dl_s = nb + r16 + shift
                    exs = []
                    for h in range(nh):
                        als_v = plsc.load_gather(rows, [r16, _full(fout + h)])
                        ex = exh(dl_s, als_v, h)
                        exs.append(ex)
                        plsc.store_scatter(outrows, [r16, _full(fout + h)],
                                           ex)
                    for col in range(fout):
                        v = plsc.load_gather(rows, [r16, _full(col)])
                        plsc.store_scatter(outrows, [r16, _full(col)],
                                           v * exs[col // f])

                pltpu.sync_copy(outrows.at[pl.ds(0, 128)],
                                accum.at[pl.ds(s * BSZ + nb, 128)])
                return cc

            lax.fori_loop(0, (nrows + 127) >> 7, init_chunk, 0)

            eb = pl.multiple_of(plsc.load_gather(bases_v, [_full(b)])[0], 8)
            cnt = plsc.load_gather(bases_v, [_full(NB + b)])[0] - eb
            nsup = (cnt + SUP - 1) >> 9

            def sup_body(blk, carry):
                pltpu.sync_copy(part.at[pl.ds(eb + blk * SUP, SUP)], partbuf)

                @plsc.parallel_loop(0, SUP // 16, unroll=2)
                def ug(g):
                    v = plsc.load_gather(partbuf, [g * 16 + iota])
                    src = jnp.clip(v & 0x1FFFF, 0, N - 1)
                    dl = jnp.clip(jnp.right_shift(v, 17), 0, BSZ - 1)
                    plsc.store_scatter(srcbuf, [g * 16 + iota], src)
                    plsc.store_scatter(scatbuf,
                                       [_full(g >> 3), (g & 7) * 16 + iota],
                                       s * BSZ + dl)

                cps = [pltpu.async_copy(
                    hprow.at[srcbuf.at[pl.ds(j * 128, 128)]],
                    rows.at[pl.ds(j * 128, 128)], sem) for j in range(4)]
                for cp in cps:
                    cp.wait()

                def cg(g):
                    r16 = g * 16 + iota
                    valid = blk * SUP + r16 < cnt
                    scat_v = plsc.load_gather(
                        scatbuf, [_full(g >> 3), (g & 7) * 16 + iota])
                    dl_s = (scat_v - s * BSZ) + shift
                    exs = []
                    for h in range(nh):
                        als_v = plsc.load_gather(rows, [r16, _full(fout + h)])
                        ex = jnp.where(valid, exh(dl_s, als_v, h), 0.0)
                        exs.append(ex)
                        plsc.store_scatter(outrows, [r16, _full(fout + h)],
                                           ex)
                    for col in range(fout):
                        v = plsc.load_gather(rows, [r16, _full(col)])
                        plsc.store_scatter(outrows, [r16, _full(col)],
                                           v * exs[col // f])

                plsc.parallel_loop(0, SUP // 16)(cg)

                scps = [pltpu.async_copy(
                    outrows.at[pl.ds(j * 128, 128)],
                    accum.at[scatbuf.at[j]], sem2, add=True)
                    for j in range(4)]
                for cp in scps:
                    cp.wait()
                return carry

            lax.fori_loop(0, nsup, sup_body, 0)

            def fin_chunk(k, cc):
                nb = jnp.minimum(k * 128, nrows - 128)
                pltpu.sync_copy(accum.at[pl.ds(s * BSZ + nb, 128)],
                                outrows.at[pl.ds(0, 128)])

                @plsc.parallel_loop(0, 8)
                def grp(g):
                    r16 = g * 16 + iota
                    dens = []
                    for h in range(nh):
                        d_v = plsc.load_gather(outrows,
                                               [r16, _full(fout + h)])
                        dens.append(jnp.where(d_v > 0, 1.0 / d_v, 0.0))
                    for col in range(fout):
                        v = plsc.load_gather(outrows, [r16, _full(col)])
                        o = v * dens[col // f] + bvecs[col // 16][col % 16]
                        if elu:
                            o = jnp.where(o > 0, o, jnp.exp(o) - 1.0)
                        plsc.store_scatter(finbuf, [r16, _full(col)], o)

                pltpu.sync_copy(finbuf,
                                out_hbm.at[pl.ds(node_base + nb, 128)])
                return cc

            lax.fori_loop(0, (nrows + 127) >> 7, fin_chunk, 0)

    return pl.kernel(
        body,
        out_type=jax.ShapeDtypeStruct((N, fout), _f32),
        mesh=_mesh,
        compiler_params=_sc_params,
        scratch_types=[
            pltpu.VMEM((BSZ, nh), _f32),
            pltpu.VMEM((16,), _f32),
            pltpu.VMEM((fout,), _f32),
            pltpu.VMEM((144,), _i32),
            pltpu.VMEM((SUP,), _i32),
            pltpu.VMEM((SUP,), _i32),
            pltpu.VMEM((4, 128), _i32),
            pltpu.VMEM((SUP, rw), _f32),
            pltpu.VMEM((SUP, acc), _f32),
            pltpu.VMEM((128, fout), _f32),
            pltpu.VMEM_SHARED((NSUB * BSZ, acc), _f32),
            pltpu.SemaphoreType.DMA,
            pltpu.SemaphoreType.DMA,
        ],
    )


_sc_layer01 = _make_sc_layer(rw=48, nh=4, f=8, acc=40, elu=True)
_sc_layer2 = _make_sc_layer(rw=32, nh=1, f=16, acc=24, elu=False)

_tc_prep0 = _make_tc(fin=1, fout=32, nh=4, rw=48, first=True)
_tc_prep1 = _make_tc(fin=32, fout=32, nh=4, rw=48, first=False)
_tc_prep2 = _make_tc(fin=32, fout=16, nh=1, rw=32, first=False)


def _blockdiag(a):
    nh, f = a.shape
    m = (jnp.arange(nh * f)[:, None] // f) == jnp.arange(nh)[None, :]
    return jnp.where(m, a.reshape(-1)[:, None], 0.0).astype(_f32)


def _pad16(g):
    v = g.reshape(-1)
    return jnp.pad(v, (0, 16 - v.shape[0]))


def kernel(x, edge_index, W_in, b_in, W0, asrc0, adst0, b0,
           W1, asrc1, adst1, b1, W2, asrc2, adst2, b2):
    src = jnp.pad(edge_index[0], (0, 128))
    dst = jnp.pad(edge_index[1], (0, 128))

    cnt = _p1(dst)
    part, bases = _p2(src, dst, cnt)

    hprow0, ald0, gmax0 = _tc_prep0(
        x, W_in.reshape(1, 16), b_in.reshape(1, 16), W0,
        _blockdiag(asrc0), _blockdiag(adst0))
    h1 = _sc_layer01(hprow0, ald0, part, bases, _pad16(gmax0), b0)

    hprow1, ald1, gmax1 = _tc_prep1(
        h1, W1, _blockdiag(asrc1), _blockdiag(adst1))
    h2 = _sc_layer01(hprow1, ald1, part, bases, _pad16(gmax1), b1)

    hprow2, ald2, gmax2 = _tc_prep2(
        h2, W2, _blockdiag(asrc2), _blockdiag(adst2))
    out = _sc_layer2(hprow2, ald2, part, bases, _pad16(gmax2), b2)
    return out
